# baseline (speedup 1.0000x reference)
import functools

import jax
import jax.numpy as jnp
from jax import lax
from jax.experimental import pallas as pl
from jax.experimental.pallas import tpu as pltpu
from jax.experimental.pallas import tpu_sc as plsc

N = 10000
D = 128
H = 64
E = 320000

NC = 2
NS = 16
NW = NC * NS
CHUNK = 64
EPW = E // NW
FULL = EPW // CHUNK
TAIL = EPW - FULL * CHUNK
CPW = FULL + 2
NPAD = 10112
ROWS_PER_TILE = NPAD // NS
DUMMY = N + 16


def _proj_body(x_ref, w1a_ref, w1b_ref, b1_ref, t_ref):
    x = x_ref[...]
    t_ref[pl.ds(0, N), :] = jnp.dot(
        x, w1a_ref[...], preferred_element_type=jnp.float32
    )
    t_ref[pl.ds(N, N), :] = (
        jnp.dot(x, w1b_ref[...], preferred_element_type=jnp.float32)
        + b1_ref[...]
    )


_proj = pl.pallas_call(
    _proj_body,
    out_shape=jax.ShapeDtypeStruct((2 * N, D), jnp.float32),
)


def _combine_body(p_ref, o_ref):
    o_ref[...] = p_ref[0, :N] + p_ref[1, :N]


_combine = pl.pallas_call(
    _combine_body,
    out_shape=jax.ShapeDtypeStruct((N, D), jnp.float32),
)


_mesh = plsc.VectorSubcoreMesh(core_axis_name="c", subcore_axis_name="s")


@functools.partial(
    pl.kernel,
    out_type=jax.ShapeDtypeStruct((NC, NPAD, D), jnp.float32),
    mesh=_mesh,
    compiler_params=pltpu.CompilerParams(needs_layout_passes=False),
    scratch_types=[
        pltpu.VMEM((2 * CHUNK,), jnp.int32),
        pltpu.VMEM((CHUNK,), jnp.int32),
        pltpu.VMEM((2 * CHUNK, D), jnp.float32),
        pltpu.VMEM((CHUNK, D), jnp.float32),
        pltpu.VMEM((2 * CHUNK,), jnp.int32),
        pltpu.VMEM((CHUNK,), jnp.int32),
        pltpu.VMEM((2 * CHUNK, D), jnp.float32),
        pltpu.VMEM((CHUNK, D), jnp.float32),
        pltpu.VMEM((384,), jnp.float32),
        pltpu.SMEM((80,), jnp.float32),
        pltpu.VMEM_SHARED((NPAD, D), jnp.float32),
        pltpu.SemaphoreType.DMA,
        pltpu.SemaphoreType.DMA,
        pltpu.SemaphoreType.DMA,
        pltpu.SemaphoreType.DMA,
        pltpu.SemaphoreType.DMA,
        pltpu.SemaphoreType.DMA,
        pltpu.SemaphoreType.DMA,
        pltpu.SemaphoreType.DMA,
        pltpu.SemaphoreType.DMA,
        pltpu.SemaphoreType.DMA,
    ],
)
def _sc_edge(ilg_hbm, ild_hbm, t_hbm, e_hbm, w2_hbm, out_hbm,
             idxgA, sdstA, abA, erowsA,
             idxgB, sdstB, abB, erowsB,
             tbuf, w2_s, acc,
             semA, semB, semEA, semEB, semIA, semIB, semDA, semDB,
             semSA, semSB):
    cid = lax.axis_index("c")
    sid = lax.axis_index("s")
    wid = sid * NC + cid

    zeros16 = jnp.zeros((16,), jnp.float32)

    def zrow(r, carry):
        for q in range(D // 16):
            erowsA[r, pl.ds(q * 16, 16)] = zeros16
        return carry

    lax.fori_loop(0, CHUNK, zrow, 0)
    for k in range(ROWS_PER_TILE // CHUNK):
        pltpu.sync_copy(
            erowsA, acc.at[pl.ds(sid * ROWS_PER_TILE + k * CHUNK, CHUNK)]
        )
    rem = ROWS_PER_TILE % CHUNK
    if rem:
        pltpu.sync_copy(
            erowsA.at[pl.ds(0, rem)],
            acc.at[pl.ds(sid * ROWS_PER_TILE
                         + (ROWS_PER_TILE // CHUNK) * CHUNK, rem)],
        )
    pltpu.sync_copy(w2_hbm, tbuf.at[pl.ds(272, 80)])
    for q in range(80 // 16):
        vq = tbuf[pl.ds(272 + q * 16, 16)]
        for i in range(16):
            w2_s[q * 16 + i] = vq[i]
    plsc.subcore_barrier()

    lane = lax.iota(jnp.int32, 16)
    b2s = w2_s[64]
    idx17 = lane * 17
    w2regs = [tbuf[pl.ds(272 + q * 16, 16)] for q in range(H // 16)]
    dummy16 = jnp.full((16,), DUMMY, jnp.int32)

    def idx_start(c, idxg, sem):
        pltpu.async_copy(ilg_hbm.at[wid, c], idxg, sem)

    def idx_wait(idxg, sem):
        pltpu.make_async_copy(ilg_hbm.at[wid, 0], idxg, sem).wait()

    def d_start(c, sdst, sem):
        pltpu.async_copy(ild_hbm.at[wid, c], sdst, sem)

    def d_wait(sdst, sem):
        pltpu.make_async_copy(ild_hbm.at[wid, 0], sdst, sem).wait()

    def ab_start(idxg, ab, sem):
        pltpu.async_copy(t_hbm.at[idxg], ab, sem)

    def ab_wait(idxg, ab, sem):
        pltpu.make_async_copy(t_hbm.at[idxg], ab, sem).wait()

    def e_start(idxg, erows, sem):
        pltpu.async_copy(e_hbm.at[idxg.at[pl.ds(0, CHUNK)]], erows, sem)

    def e_wait(idxg, erows, sem):
        pltpu.make_async_copy(
            e_hbm.at[idxg.at[pl.ds(0, CHUNK)]], erows, sem
        ).wait()

    def s_start(erows, sdst, sem):
        pltpu.async_copy(erows, acc.at[sdst], sem, add=True)

    def s_wait(erows, sdst, sem):
        pltpu.make_async_copy(erows, acc.at[sdst], sem).wait()

    def compute(ab, erows):
        def group_body(g, gc):
            for ii in range(16):
                e = g * 16 + ii
                acc_v = None
                for q in range(H // 16):
                    av = ab[e, pl.ds(q * 16, 16)]
                    bv = ab[CHUNK + e, pl.ds(q * 16, 16)]
                    hv = jnp.maximum(av + bv, 0.0) * w2regs[q]
                    acc_v = hv if acc_v is None else acc_v + hv
                plsc.store_scatter(tbuf, [idx17 + ii], acc_v)
            lvec = jnp.full((16,), 0.0, jnp.float32)
            for j in range(16):
                lvec = lvec + plsc.load_gather(tbuf, [lane + j * 17])
            mvec = 1.0 / (1.0 + jnp.exp(-(lvec + b2s)))
            for ii in range(16):
                m = mvec[ii]
                e = g * 16 + ii
                for q in range(D // 16):
                    erows[e, pl.ds(q * 16, 16)] = (
                        erows[e, pl.ds(q * 16, 16)] * m
                    )
            return gc

        lax.fori_loop(0, CHUNK // 16, group_body, 0)

    idx_start(0, idxgA, semIA)
    idx_wait(idxgA, semIA)
    ab_start(idxgA, abA, semA)
    e_start(idxgA, erowsA, semEA)
    idx_start(1, idxgB, semIB)
    d_start(0, sdstA, semDA)
    pltpu.async_copy(erowsA, acc.at[pl.ds(N, CHUNK)], semSA)
    pltpu.async_copy(erowsB, acc.at[pl.ds(N, CHUNK)], semSB)

    def phase(c_pre, idxgX, sdstX, abX, erowsX, semX, semEX, semIX, semDX,
              semSX, idxgY, sdstY, abY, erowsY, semY, semEY, semIY, semDY,
              semSY):
        ab_wait(idxgX, abX, semX)
        e_wait(idxgX, erowsX, semEX)
        idx_wait(idxgY, semIY)
        ab_start(idxgY, abY, semY)
        idx_start(c_pre, idxgX, semIX)
        compute(abX, erowsX)
        d_wait(sdstX, semDX)
        s_start(erowsX, sdstX, semSX)
        s_wait(erowsY, sdstY, semSY)
        d_start(c_pre - 1, sdstY, semDY)
        e_start(idxgY, erowsY, semEY)

    def pair_body(it, carry):
        c0 = 2 * it
        phase(c0 + 2, idxgA, sdstA, abA, erowsA, semA, semEA, semIA, semDA,
              semSA, idxgB, sdstB, abB, erowsB, semB, semEB, semIB, semDB,
              semSB)
        phase(c0 + 3, idxgB, sdstB, abB, erowsB, semB, semEB, semIB, semDB,
              semSB, idxgA, sdstA, abA, erowsA, semA, semEA, semIA, semDA,
              semSA)
        return carry

    lax.fori_loop(0, FULL // 2, pair_body, 0)

    ab_wait(idxgA, abA, semA)
    e_wait(idxgA, erowsA, semEA)
    idx_wait(idxgB, semIB)
    compute(abA, erowsA)
    d_wait(sdstA, semDA)
    for q in range(CHUNK // 16):
        if q * 16 >= TAIL:
            sdstA[pl.ds(q * 16, 16)] = dummy16
    s_start(erowsA, sdstA, semSA)
    s_wait(erowsB, sdstB, semSB)
    s_wait(erowsA, sdstA, semSA)

    plsc.subcore_barrier()

    r0 = sid * ROWS_PER_TILE
    pltpu.sync_copy(acc.at[pl.ds(r0, ROWS_PER_TILE)],
                    out_hbm.at[cid, pl.ds(r0, ROWS_PER_TILE)])


@jax.jit
def kernel(embed, edge_index, W1, b1, W2, b2):
    pad = jnp.zeros((D, D - H), jnp.float32)
    w1a = jnp.concatenate([W1[:D], pad], axis=1)
    w1b = jnp.concatenate([W1[D:], pad], axis=1)
    b1p = jnp.concatenate([b1, jnp.zeros((D - H,), jnp.float32)])
    t = _proj(embed, w1a, w1b, b1p.reshape(1, D))
    w2full = jnp.concatenate(
        [W2[:, 0], b2, jnp.zeros((15,), jnp.float32)]
    )
    padn = CPW * CHUNK - EPW
    src = jnp.pad(edge_index[0].reshape(NW, EPW), ((0, 0), (0, padn)))
    dst = jnp.pad(edge_index[1].reshape(NW, EPW), ((0, 0), (0, padn)))
    srcc = src.reshape(NW, CPW, CHUNK)
    dstc = dst.reshape(NW, CPW, CHUNK)
    ilg = jnp.concatenate([srcc, dstc + N], axis=2)
    partial = _sc_edge(ilg, dstc, t, embed, w2full)
    return _combine(partial)
